# baseline (speedup 1.0000x reference)
import functools

import jax
import jax.numpy as jnp
from jax import lax
from jax.experimental import pallas as pl
from jax.experimental.pallas import tpu as pltpu
from jax.experimental.pallas import tpu_sc as plsc

_NODE_VOCABS = (120, 10, 12)
_EDGE_VOCABS = (6, 3)
_MASK_RATE = 0.15

_NC = 2
_NS = 16
_L = 16

_CH = 112
_HH = 32
_IB = 64


def _ceil_to(x, m):
    return (x + m - 1) // m * m


_NB = 2


def _msg_agg_kernel(NP, nchunk, rows_per_sub):
    mesh = plsc.VectorSubcoreMesh(core_axis_name="c", subcore_axis_name="s")

    nblk = nchunk // _IB

    def run_half(h, C, agg_out, sci_r, dst_r, zer, s,
                 scib, dstb, sidx, cidx, rows, crows, aggs, semg, sems):
        pltpu.sync_copy(zer, aggs.at[pl.ds(s * rows_per_sub, rows_per_sub)])
        row0 = s * nchunk
        plsc.subcore_barrier()

        def decode_issue(k, b):
            for g in range(_CH // _L):
                v = scib[k, pl.ds(g * _L, _L)]
                sidx[b][pl.ds(g * _L, _L)] = lax.shift_right_logical(v, 5)
                cidx[b][pl.ds(g * _L, _L)] = lax.bitwise_and(v, 31)
            pltpu.async_copy(h.at[sidx[b]], rows[b], semg[b])
            pltpu.async_copy(C.at[cidx[b]], crows[b], semg[b])

        def block(bi, carry):
            @pl.when(bi > 0)
            def _():
                for b in range(_NB):
                    pltpu.make_async_copy(
                        rows[b], aggs.at[dstb.at[0]], sems[b]).wait()

            r0 = row0 + bi * _IB
            pltpu.sync_copy(sci_r.at[pl.ds(r0, _IB)], scib)
            pltpu.sync_copy(dst_r.at[pl.ds(r0, _IB)], dstb)
            decode_issue(0, 0)

            def group(gi, c1):
                for b in range(_NB):
                    k = gi * _NB + b
                    kf = k + 1
                    bf = (b + 1) % _NB

                    @pl.when(kf < _IB)
                    def _():
                        @pl.when(kf >= _NB)
                        def _():
                            pltpu.make_async_copy(
                                rows[bf], aggs.at[dstb.at[k]], sems[bf]).wait()
                        decode_issue(kf, bf)

                    pltpu.make_async_copy(h.at[sidx[b]], rows[b], semg[b]).wait()
                    pltpu.make_async_copy(C.at[cidx[b]], crows[b], semg[b]).wait()

                    def jbody(j, c2):
                        for t in range(2):
                            a = rows[b][j, pl.ds(t * _L, _L)]
                            cc = crows[b][j, pl.ds(t * _L, _L)]
                            rows[b][j, pl.ds(t * _L, _L)] = jnp.maximum(
                                a + cc, 0.0)
                        return c2

                    lax.fori_loop(0, _CH, jbody, 0, unroll=4)
                    pltpu.async_copy(rows[b], aggs.at[dstb.at[k]], sems[b])
                return c1

            lax.fori_loop(0, _IB // _NB, group, 0)
            return carry

        lax.fori_loop(0, nblk, block, 0)
        for b in range(_NB):
            pltpu.make_async_copy(rows[b], aggs.at[dstb.at[0]], sems[b]).wait()
        plsc.subcore_barrier()
        sl = pl.ds(s * rows_per_sub, rows_per_sub)
        pltpu.sync_copy(aggs.at[sl], agg_out.at[sl])

    @functools.partial(
        pl.kernel,
        out_type=(
            jax.ShapeDtypeStruct((NP, _HH), jnp.float32),
            jax.ShapeDtypeStruct((NP, _HH), jnp.float32),
        ),
        mesh=mesh,
        scratch_types=(
            pltpu.VMEM((_IB, _CH), jnp.int32),
            pltpu.VMEM((_IB, _CH), jnp.int32),
            pltpu.VMEM((_CH,), jnp.int32),
            pltpu.VMEM((_CH,), jnp.int32),
            pltpu.VMEM((_CH,), jnp.int32),
            pltpu.VMEM((_CH,), jnp.int32),
            pltpu.VMEM((_CH, _HH), jnp.float32),
            pltpu.VMEM((_CH, _HH), jnp.float32),
            pltpu.VMEM((_CH, _HH), jnp.float32),
            pltpu.VMEM((_CH, _HH), jnp.float32),
            pltpu.VMEM_SHARED((NP, _HH), jnp.float32),
            pltpu.SemaphoreType.DMA,
            pltpu.SemaphoreType.DMA,
            pltpu.SemaphoreType.DMA,
            pltpu.SemaphoreType.DMA,
        ),
        compiler_params=pltpu.CompilerParams(use_tc_tiling_on_sc=False),
    )
    def kern(hA, hB, CA, CB, sci_r, dst_r, zer,
             aggA, aggB,
             scib, dstb, sidx0, sidx1, cidx0, cidx1,
             rows0, rows1, crows0, crows1, aggs,
             semg0, semg1, sems0, sems1):
        c = lax.axis_index("c")
        s = lax.axis_index("s")
        sidx = (sidx0, sidx1)
        cidx = (cidx0, cidx1)
        rows = (rows0, rows1)
        crows = (crows0, crows1)
        semg = (semg0, semg1)
        sems = (sems0, sems1)

        @pl.when(c == 0)
        def _():
            run_half(hA, CA, aggA, sci_r, dst_r, zer, s,
                     scib, dstb, sidx, cidx, rows, crows, aggs, semg, sems)

        @pl.when(c == 1)
        def _():
            run_half(hB, CB, aggB, sci_r, dst_r, zer, s,
                     scib, dstb, sidx, cidx, rows, crows, aggs, semg, sems)

    return kern


def kernel(x, edge_index, edge_attr, params):
    N = x.shape[0]
    E = edge_attr.shape[0]
    H = params["node_emb"][0].shape[1]

    mkey = jax.random.key(42)
    node_mask = jax.random.uniform(jax.random.fold_in(mkey, 0), (N,)) < _MASK_RATE
    edge_mask = jax.random.uniform(jax.random.fold_in(mkey, 1), (E,)) < _MASK_RATE

    nfill = jnp.array([v - 1 for v in _NODE_VOCABS], dtype=x.dtype)
    efill = jnp.array([v - 1 for v in _EDGE_VOCABS], dtype=edge_attr.dtype)
    xm = jnp.where(node_mask[:, None], nfill[None, :], x)
    eam = jnp.where(edge_mask[:, None], efill[None, :], edge_attr)

    h = jnp.zeros((N, H), jnp.float32)
    for i, t in enumerate(params["node_emb"]):
        h = h + jnp.take(t, xm[:, i], axis=0)

    T0, T1 = params["edge_emb"]
    C = (T0[:, None, :] + T1[None, :, :]).reshape(
        _EDGE_VOCABS[0] * _EDGE_VOCABS[1], H)
    ci = eam[:, 0] * _EDGE_VOCABS[1] + eam[:, 1]

    per_sub = _ceil_to(-(-E // _NS), _CH * _IB)
    EP = per_sub * _NS
    nchunk = per_sub // _CH
    rows_per_sub = _ceil_to(-(-(N + 1) // _NS), 8)
    NP = rows_per_sub * _NS

    src = edge_index[0]
    dst = edge_index[1]
    pad_e = EP - E
    sci = src * 32 + ci
    sci_p = jnp.pad(sci, (0, pad_e)).reshape(EP // _CH, _CH)
    dst_p = jnp.pad(dst, (0, pad_e), constant_values=N).reshape(EP // _CH, _CH)
    zer = jnp.zeros((rows_per_sub, _HH), jnp.float32)
    CA, CB = C[:, :_HH], C[:, _HH:]

    sc_msg_agg = _msg_agg_kernel(NP, nchunk, rows_per_sub)

    for layer in params["layers"]:
        hp = jnp.pad(h, ((0, NP - N), (0, 0)))
        aggA, aggB = sc_msg_agg(hp[:, :_HH], hp[:, _HH:], CA, CB,
                                sci_p, dst_p, zer)
        agg = jnp.concatenate([aggA[:N], aggB[:N]], axis=1)
        z = h + agg
        z = jax.nn.relu(z @ layer["lin1"]["W"] + layer["lin1"]["b"])
        h = z @ layer["lin2"]["W"] + layer["lin2"]["b"]

    Wn = jnp.concatenate([hd["W"] for hd in params["node_heads"]], axis=1)
    bn = jnp.concatenate([hd["b"] for hd in params["node_heads"]])
    node_pred = h @ Wn + bn

    We = jnp.concatenate([hd["W"] for hd in params["edge_heads"]], axis=1)
    be = jnp.concatenate([hd["b"] for hd in params["edge_heads"]])
    Q = h @ We
    edge_pred = jnp.take(Q, src, axis=0) + jnp.take(Q, dst, axis=0) + be

    return node_pred, edge_pred, node_mask, edge_mask


# speedup vs baseline: 1.0047x; 1.0047x over previous
import functools

import jax
import jax.numpy as jnp
from jax import lax
from jax.experimental import pallas as pl
from jax.experimental.pallas import tpu as pltpu
from jax.experimental.pallas import tpu_sc as plsc

_NODE_VOCABS = (120, 10, 12)
_EDGE_VOCABS = (6, 3)
_MASK_RATE = 0.15

_NC = 2
_NS = 16
_L = 16

_CH = 112
_HH = 32
_IB = 32


def _ceil_to(x, m):
    return (x + m - 1) // m * m


_NB = 2


def _msg_agg_kernel(NP, nchunk, rows_per_sub):
    mesh = plsc.VectorSubcoreMesh(core_axis_name="c", subcore_axis_name="s")

    nblk = nchunk // _IB

    def run_half(h, C, agg_out, src_r, ci_r, dst_r, zer, s,
                 srcb, cib, dstb, rows, crows, aggs, semg, sems):
        pltpu.sync_copy(zer, aggs.at[pl.ds(s * rows_per_sub, rows_per_sub)])
        row0 = s * nchunk
        plsc.subcore_barrier()

        def issue_gathers(k, b):
            pltpu.async_copy(h.at[srcb.at[k]], rows[b], semg[b])
            pltpu.async_copy(C.at[cib.at[k]], crows[b], semg[b])

        def block(bi, carry):
            @pl.when(bi > 0)
            def _():
                for b in range(_NB):
                    pltpu.make_async_copy(
                        rows[b], aggs.at[dstb.at[0]], sems[b]).wait()

            r0 = row0 + bi * _IB
            pltpu.sync_copy(src_r.at[pl.ds(r0, _IB)], srcb)
            pltpu.sync_copy(ci_r.at[pl.ds(r0, _IB)], cib)
            pltpu.sync_copy(dst_r.at[pl.ds(r0, _IB)], dstb)
            issue_gathers(0, 0)

            def group(gi, c1):
                for b in range(_NB):
                    k = gi * _NB + b
                    kf = k + 1
                    bf = (b + 1) % _NB

                    @pl.when(kf < _IB)
                    def _():
                        @pl.when(kf >= _NB)
                        def _():
                            pltpu.make_async_copy(
                                rows[bf], aggs.at[dstb.at[k]], sems[bf]).wait()
                        issue_gathers(kf, bf)

                    pltpu.make_async_copy(
                        h.at[srcb.at[k]], rows[b], semg[b]).wait()
                    pltpu.make_async_copy(
                        C.at[cib.at[k]], crows[b], semg[b]).wait()

                    def jbody(j, c2):
                        for t in range(2):
                            a = rows[b][j, pl.ds(t * _L, _L)]
                            cc = crows[b][j, pl.ds(t * _L, _L)]
                            rows[b][j, pl.ds(t * _L, _L)] = jnp.maximum(
                                a + cc, 0.0)
                        return c2

                    lax.fori_loop(0, _CH, jbody, 0, unroll=4)
                    pltpu.async_copy(rows[b], aggs.at[dstb.at[k]], sems[b],
                                     add=True)
                return c1

            lax.fori_loop(0, _IB // _NB, group, 0)
            return carry

        lax.fori_loop(0, nblk, block, 0)
        for b in range(_NB):
            pltpu.make_async_copy(rows[b], aggs.at[dstb.at[0]], sems[b]).wait()
        plsc.subcore_barrier()
        sl = pl.ds(s * rows_per_sub, rows_per_sub)
        pltpu.sync_copy(aggs.at[sl], agg_out.at[sl])

    @functools.partial(
        pl.kernel,
        out_type=(
            jax.ShapeDtypeStruct((NP, _HH), jnp.float32),
            jax.ShapeDtypeStruct((NP, _HH), jnp.float32),
        ),
        mesh=mesh,
        scratch_types=(
            pltpu.VMEM((_IB, _CH), jnp.int32),
            pltpu.VMEM((_IB, _CH), jnp.int32),
            pltpu.VMEM((_IB, _CH), jnp.int32),
            pltpu.VMEM((_CH, _HH), jnp.float32),
            pltpu.VMEM((_CH, _HH), jnp.float32),
            pltpu.VMEM((_CH, _HH), jnp.float32),
            pltpu.VMEM((_CH, _HH), jnp.float32),
            pltpu.VMEM_SHARED((NP, _HH), jnp.float32),
            pltpu.SemaphoreType.DMA,
            pltpu.SemaphoreType.DMA,
            pltpu.SemaphoreType.DMA,
            pltpu.SemaphoreType.DMA,
        ),
        compiler_params=pltpu.CompilerParams(use_tc_tiling_on_sc=False),
    )
    def kern(hA, hB, CA, CB, src_r, ci_r, dst_r, zer,
             aggA, aggB,
             srcb, cib, dstb,
             rows0, rows1, crows0, crows1, aggs,
             semg0, semg1, sems0, sems1):
        c = lax.axis_index("c")
        s = lax.axis_index("s")
        rows = (rows0, rows1)
        crows = (crows0, crows1)
        semg = (semg0, semg1)
        sems = (sems0, sems1)

        @pl.when(c == 0)
        def _():
            run_half(hA, CA, aggA, src_r, ci_r, dst_r, zer, s,
                     srcb, cib, dstb, rows, crows, aggs, semg, sems)

        @pl.when(c == 1)
        def _():
            run_half(hB, CB, aggB, src_r, ci_r, dst_r, zer, s,
                     srcb, cib, dstb, rows, crows, aggs, semg, sems)

    return kern


def kernel(x, edge_index, edge_attr, params):
    N = x.shape[0]
    E = edge_attr.shape[0]
    H = params["node_emb"][0].shape[1]

    mkey = jax.random.key(42)
    node_mask = jax.random.uniform(jax.random.fold_in(mkey, 0), (N,)) < _MASK_RATE
    edge_mask = jax.random.uniform(jax.random.fold_in(mkey, 1), (E,)) < _MASK_RATE

    nfill = jnp.array([v - 1 for v in _NODE_VOCABS], dtype=x.dtype)
    efill = jnp.array([v - 1 for v in _EDGE_VOCABS], dtype=edge_attr.dtype)
    xm = jnp.where(node_mask[:, None], nfill[None, :], x)
    eam = jnp.where(edge_mask[:, None], efill[None, :], edge_attr)

    h = jnp.zeros((N, H), jnp.float32)
    for i, t in enumerate(params["node_emb"]):
        h = h + jnp.take(t, xm[:, i], axis=0)

    T0, T1 = params["edge_emb"]
    C = (T0[:, None, :] + T1[None, :, :]).reshape(
        _EDGE_VOCABS[0] * _EDGE_VOCABS[1], H)
    ci = eam[:, 0] * _EDGE_VOCABS[1] + eam[:, 1]

    per_sub = _ceil_to(-(-E // _NS), _CH * _IB)
    EP = per_sub * _NS
    nchunk = per_sub // _CH
    rows_per_sub = _ceil_to(-(-(N + 1) // _NS), 8)
    NP = rows_per_sub * _NS

    src = edge_index[0]
    dst = edge_index[1]
    pad_e = EP - E
    src_p = jnp.pad(src, (0, pad_e)).reshape(EP // _CH, _CH)
    ci_p = jnp.pad(ci, (0, pad_e)).reshape(EP // _CH, _CH)
    dst_p = jnp.pad(dst, (0, pad_e), constant_values=N).reshape(EP // _CH, _CH)
    zer = jnp.zeros((rows_per_sub, _HH), jnp.float32)
    CA, CB = C[:, :_HH], C[:, _HH:]

    sc_msg_agg = _msg_agg_kernel(NP, nchunk, rows_per_sub)

    for layer in params["layers"]:
        hp = jnp.pad(h, ((0, NP - N), (0, 0)))
        aggA, aggB = sc_msg_agg(hp[:, :_HH], hp[:, _HH:], CA, CB,
                                src_p, ci_p, dst_p, zer)
        agg = jnp.concatenate([aggA[:N], aggB[:N]], axis=1)
        z = h + agg
        z = jax.nn.relu(z @ layer["lin1"]["W"] + layer["lin1"]["b"])
        h = z @ layer["lin2"]["W"] + layer["lin2"]["b"]

    Wn = jnp.concatenate([hd["W"] for hd in params["node_heads"]], axis=1)
    bn = jnp.concatenate([hd["b"] for hd in params["node_heads"]])
    node_pred = h @ Wn + bn

    We = jnp.concatenate([hd["W"] for hd in params["edge_heads"]], axis=1)
    be = jnp.concatenate([hd["b"] for hd in params["edge_heads"]])
    Q = h @ We
    edge_pred = jnp.take(Q, src, axis=0) + jnp.take(Q, dst, axis=0) + be

    return node_pred, edge_pred, node_mask, edge_mask
